# Initial kernel scaffold; baseline (speedup 1.0000x reference)
import jax
import jax.numpy as jnp
from jax.experimental import pallas as pl


def kernel(x, edge_index, K_DD, K_DA, W1, b1, W2, b2):
    raise NotImplementedError("write your pallas kernel here")


# speedup vs baseline: 2.2469x; 2.2469x over previous
import functools

import jax
import jax.numpy as jnp
from jax import lax
from jax.experimental import pallas as pl
from jax.experimental.pallas import tpu as pltpu
from jax.experimental.pallas import tpu_sc as plsc

NB = 128
BS = 8
NSC = 2
NTILE = 16
LANES = 16
CB = 128


def _fill1(ref, n, val):
    def body(i, _):
        ref[pl.ds(i * LANES, LANES)] = jnp.full((LANES,), val, jnp.float32)
        return None
    lax.fori_loop(0, n // LANES, body, None)


def _zero2(ref, nrows, ncols):
    def body(i, _):
        for j in range(ncols // LANES):
            ref[i, pl.ds(j * LANES, LANES)] = jnp.zeros((LANES,), jnp.float32)
        return None
    lax.fori_loop(0, nrows, body, None)


def _edge_pass(table_h, src_h, dst_h, k_h, c, s, src_v, dst_v, k_v, rows,
               gsem, agg_s, nblk, d, ones_v=None, deg_s=None):
    def block(bo, _):
        pltpu.sync_copy(src_h.at[s, bo], src_v)
        pltpu.sync_copy(dst_h.at[s, bo], dst_v)
        pltpu.sync_copy(k_h.at[c, s, bo], k_v)

        def chunk(i, _):
            pltpu.async_copy(table_h.at[src_v.at[i]], rows, gsem).wait()

            def mul(g, _):
                kvec = k_v[i, pl.ds(g * LANES, LANES)]
                for lane in range(LANES):
                    e = g * LANES + lane
                    kv = kvec[lane]
                    for j in range(d // LANES):
                        rows[e, pl.ds(j * LANES, LANES)] = (
                            rows[e, pl.ds(j * LANES, LANES)] * kv)
                return None
            lax.fori_loop(0, NB // LANES, mul, None)

            pltpu.sync_copy(rows, agg_s.at[dst_v.at[i]], add=True)
            if deg_s is not None:
                pltpu.sync_copy(ones_v, deg_s.at[dst_v.at[i]], add=True)
            return None

        lax.fori_loop(0, BS, chunk, None)
        return None

    lax.fori_loop(0, nblk, block, None)


def _copyout(agg_s, aggbuf, invbuf, out_ref, r0, rpt, d):
    for cb in range(rpt // CB):
        base = r0 + cb * CB
        pltpu.sync_copy(agg_s.at[pl.ds(base, CB)], aggbuf)

        def row(g, _, cb=cb):
            ivec = invbuf[pl.ds(cb * CB + g * LANES, LANES)]
            for lane in range(LANES):
                r = g * LANES + lane
                iv = ivec[lane]
                for j in range(d // LANES):
                    aggbuf[r, pl.ds(j * LANES, LANES)] = (
                        aggbuf[r, pl.ds(j * LANES, LANES)] * iv)
            return None
        lax.fori_loop(0, CB // LANES, row, None)
        pltpu.sync_copy(aggbuf, out_ref.at[pl.ds(base, CB)])


def _make_stage1(npad, d, nblk):
    rpt = npad // NTILE
    mesh = plsc.VectorSubcoreMesh(core_axis_name="c", subcore_axis_name="s",
                                  num_cores=NSC, num_subcores=NTILE)

    @functools.partial(
        pl.kernel,
        out_type=(jax.ShapeDtypeStruct((NSC, npad, d), jnp.float32),
                  jax.ShapeDtypeStruct((npad,), jnp.float32)),
        mesh=mesh,
        scratch_types=[
            pltpu.VMEM((BS, NB), jnp.int32),
            pltpu.VMEM((BS, NB), jnp.int32),
            pltpu.VMEM((BS, NB), jnp.float32),
            pltpu.VMEM((NB, d), jnp.float32),
            pltpu.VMEM((NB,), jnp.float32),
            pltpu.VMEM((CB, d), jnp.float32),
            pltpu.VMEM((rpt,), jnp.float32),
            pltpu.VMEM_SHARED((npad, d), jnp.float32),
            pltpu.VMEM_SHARED((npad,), jnp.float32),
            pltpu.SemaphoreType.DMA,
        ],
    )
    def stage1(x_h, src_h, dst_h, k_h, h1_h, inv_h,
               src_v, dst_v, k_v, rows, ones_v, aggbuf, invbuf,
               agg_s, deg_s, gsem):
        c = lax.axis_index("c")
        s = lax.axis_index("s")
        r0 = s * rpt

        _fill1(ones_v, NB, 1.0)
        _zero2(aggbuf, CB, d)
        for cb in range(rpt // CB):
            pltpu.sync_copy(aggbuf, agg_s.at[pl.ds(r0 + cb * CB, CB)])
        _fill1(invbuf, rpt, 0.0)
        pltpu.sync_copy(invbuf, deg_s.at[pl.ds(r0, rpt)])
        plsc.subcore_barrier()

        _edge_pass(x_h, src_h, dst_h, k_h, c, s, src_v, dst_v, k_v, rows,
                   gsem, agg_s, nblk, d, ones_v=ones_v, deg_s=deg_s)
        plsc.subcore_barrier()

        pltpu.sync_copy(deg_s.at[pl.ds(r0, rpt)], invbuf)

        def inv(i, _):
            dv = invbuf[pl.ds(i * LANES, LANES)]
            invbuf[pl.ds(i * LANES, LANES)] = 1.0 / jnp.maximum(dv, 1.0)
            return None
        lax.fori_loop(0, rpt // LANES, inv, None)

        @pl.when(c == 0)
        def _():
            pltpu.sync_copy(invbuf, inv_h.at[pl.ds(r0, rpt)])

        _copyout(agg_s, aggbuf, invbuf, h1_h.at[c], r0, rpt, d)

    return stage1


def _make_stage2(npad, d, nblk):
    rpt = npad // NTILE
    mesh = plsc.VectorSubcoreMesh(core_axis_name="c", subcore_axis_name="s",
                                  num_cores=NSC, num_subcores=NTILE)

    @functools.partial(
        pl.kernel,
        out_type=jax.ShapeDtypeStruct((NSC, 2, npad, d), jnp.float32),
        mesh=mesh,
        scratch_types=[
            pltpu.VMEM((BS, NB), jnp.int32),
            pltpu.VMEM((BS, NB), jnp.int32),
            pltpu.VMEM((BS, NB), jnp.float32),
            pltpu.VMEM((NB, d), jnp.float32),
            pltpu.VMEM((CB, d), jnp.float32),
            pltpu.VMEM((rpt,), jnp.float32),
            pltpu.VMEM_SHARED((npad, d), jnp.float32),
            pltpu.SemaphoreType.DMA,
        ],
    )
    def stage2(t0_h, t1_h, src_h, dst_h, k_h, inv_h, q_h,
               src_v, dst_v, k_v, rows, aggbuf, invbuf, agg_s, gsem):
        c = lax.axis_index("c")
        s = lax.axis_index("s")
        r0 = s * rpt

        pltpu.sync_copy(inv_h.at[pl.ds(r0, rpt)], invbuf)

        for p, table_h in enumerate((t0_h, t1_h)):
            _zero2(aggbuf, CB, d)
            for cb in range(rpt // CB):
                pltpu.sync_copy(aggbuf, agg_s.at[pl.ds(r0 + cb * CB, CB)])
            plsc.subcore_barrier()

            _edge_pass(table_h, src_h, dst_h, k_h, c, s, src_v, dst_v, k_v,
                       rows, gsem, agg_s, nblk, d)
            plsc.subcore_barrier()

            _copyout(agg_s, aggbuf, invbuf, q_h.at[c, p], r0, rpt, d)

    return stage2


def _mlp_block(q0, q1, q2, q3, w1, b1, w2, b2, o_ref):
    d = q0.shape[1]
    acc = jnp.dot(q0[...], w1[0:d, :], preferred_element_type=jnp.float32)
    acc += jnp.dot(q1[...], w1[d:2 * d, :], preferred_element_type=jnp.float32)
    acc += jnp.dot(q2[...], w1[2 * d:3 * d, :],
                   preferred_element_type=jnp.float32)
    acc += jnp.dot(q3[...], w1[3 * d:4 * d, :],
                   preferred_element_type=jnp.float32)
    h = jnp.maximum(acc + b1[...], 0.0)
    o = jnp.dot(h, w2[...], preferred_element_type=jnp.float32) + b2[...]
    ss = jnp.sum(o * o, axis=-1, keepdims=True)
    o_ref[...] = o / jnp.maximum(jnp.sqrt(ss), 1e-12)


def _make_mlp(npad, d, hidden, out):
    rb = 640
    grid = npad // rb
    return pl.pallas_call(
        _mlp_block,
        grid=(grid,),
        in_specs=[
            pl.BlockSpec((rb, d), lambda i: (i, 0)),
            pl.BlockSpec((rb, d), lambda i: (i, 0)),
            pl.BlockSpec((rb, d), lambda i: (i, 0)),
            pl.BlockSpec((rb, d), lambda i: (i, 0)),
            pl.BlockSpec((4 * d, hidden), lambda i: (0, 0)),
            pl.BlockSpec((1, hidden), lambda i: (0, 0)),
            pl.BlockSpec((hidden, out), lambda i: (0, 0)),
            pl.BlockSpec((1, out), lambda i: (0, 0)),
        ],
        out_specs=pl.BlockSpec((rb, out), lambda i: (i, 0)),
        out_shape=jax.ShapeDtypeStruct((npad, out), jnp.float32),
    )


@jax.jit
def kernel(x, edge_index, K_DD, K_DA, W1, b1, W2, b2):
    n, d = x.shape
    e = edge_index.shape[1]
    hidden = W1.shape[1]
    out = W2.shape[1]
    npad = ((n + 256) // 256) * 256

    ept = NB * BS * -(-e // (NB * BS * NTILE))
    nblk = ept // (NB * BS)
    e_pad = ept * NTILE
    pad = e_pad - e
    src = jnp.pad(edge_index[0], (0, pad)).reshape(NTILE, nblk, BS, NB)
    dst = jnp.pad(edge_index[1], (0, pad),
                  constant_values=n).reshape(NTILE, nblk, BS, NB)
    kdd = jnp.pad(K_DD, ((0, 0), (0, pad))).reshape(NSC, NTILE, nblk, BS, NB)
    kda = jnp.pad(K_DA, ((0, 0), (0, pad))).reshape(NSC, NTILE, nblk, BS, NB)

    stage1 = _make_stage1(npad, d, nblk)
    h1, inv = stage1(x, src, dst, kdd)

    stage2 = _make_stage2(npad, d, nblk)
    q = stage2(h1[0], h1[1], src, dst, kda, inv)

    mlp = _make_mlp(npad, d, hidden, out)
    res = mlp(q[0, 0], q[0, 1], q[1, 0], q[1, 1], W1,
              b1.reshape(1, hidden), W2, b2.reshape(1, out))
    return res[:n]


# speedup vs baseline: 2.3719x; 1.0556x over previous
import functools

import jax
import jax.numpy as jnp
from jax import lax
from jax.experimental import pallas as pl
from jax.experimental.pallas import tpu as pltpu
from jax.experimental.pallas import tpu_sc as plsc

NB = 128
BS = 8
NSC = 2
NTILE = 16
LANES = 16
CB = 64


def _fill1(ref, n, val):
    def body(i, _):
        ref[pl.ds(i * LANES, LANES)] = jnp.full((LANES,), val, jnp.float32)
        return None
    lax.fori_loop(0, n // LANES, body, None)


def _zero2(ref, nrows, ncols):
    def body(i, _):
        for j in range(ncols // LANES):
            ref[i, pl.ds(j * LANES, LANES)] = jnp.zeros((LANES,), jnp.float32)
        return None
    lax.fori_loop(0, nrows, body, None)


def _edge_pass(table_h, src_h, dst_h, k_h, c, s, src_v, dst_v, k_v,
               rows2, gsem, ssem2, agg_s, nblk, d, ones_v=None, deg_s=None):
    def mul(rows, ch):
        def body(e, _):
            base = (e // LANES) * LANES
            wvec = k_v[ch, pl.ds(base, LANES)]
            kv = wvec.at[jnp.full((LANES,), e - base, jnp.int32)].get(
                mode="promise_in_bounds")
            for j in range(d // LANES):
                rows[e, pl.ds(j * LANES, LANES)] = (
                    rows[e, pl.ds(j * LANES, LANES)] * kv)
            return None
        lax.fori_loop(0, NB, body, None)

    def block(bo, _):
        pltpu.sync_copy(src_h.at[s, bo], src_v)
        pltpu.sync_copy(dst_h.at[s, bo], dst_v)
        pltpu.sync_copy(k_h.at[c, s, bo], k_v)

        gd = pltpu.async_copy(table_h.at[src_v.at[0]], rows2[0], gsem)
        sd = [None, None]
        dd = [None, None]
        for ch in range(BS):
            b = ch & 1
            gd.wait()
            if ch + 1 < BS:
                if sd[1 - b] is not None:
                    sd[1 - b].wait()
                    if dd[1 - b] is not None:
                        dd[1 - b].wait()
                gd = pltpu.async_copy(table_h.at[src_v.at[ch + 1]],
                                      rows2[1 - b], gsem)
            mul(rows2[b], ch)
            sd[b] = pltpu.async_copy(rows2[b], agg_s.at[dst_v.at[ch]],
                                     ssem2[b], add=True)
            if deg_s is not None:
                dd[b] = pltpu.async_copy(ones_v, deg_s.at[dst_v.at[ch]],
                                         ssem2[b], add=True)
        for b in range(2):
            if sd[b] is not None:
                sd[b].wait()
            if dd[b] is not None:
                dd[b].wait()
        return None

    lax.fori_loop(0, nblk, block, None)


def _copyout(agg_s, aggbuf, invbuf, out_ref, r0, rpt, d):
    for cb in range(rpt // CB):
        base = r0 + cb * CB
        pltpu.sync_copy(agg_s.at[pl.ds(base, CB)], aggbuf)

        def row(r, _, cb=cb):
            base = (r // LANES) * LANES
            ivec = invbuf[pl.ds(cb * CB + base, LANES)]
            iv = ivec.at[jnp.full((LANES,), r - base, jnp.int32)].get(
                mode="promise_in_bounds")
            for j in range(d // LANES):
                aggbuf[r, pl.ds(j * LANES, LANES)] = (
                    aggbuf[r, pl.ds(j * LANES, LANES)] * iv)
            return None
        lax.fori_loop(0, CB, row, None)
        pltpu.sync_copy(aggbuf, out_ref.at[pl.ds(base, CB)])


def _make_stage1(npad, d, nblk):
    rpt = npad // NTILE
    mesh = plsc.VectorSubcoreMesh(core_axis_name="c", subcore_axis_name="s",
                                  num_cores=NSC, num_subcores=NTILE)

    @functools.partial(
        pl.kernel,
        out_type=(jax.ShapeDtypeStruct((NSC, npad, d), jnp.float32),
                  jax.ShapeDtypeStruct((npad,), jnp.float32)),
        mesh=mesh,
        scratch_types=[
            pltpu.VMEM((BS, NB), jnp.int32),
            pltpu.VMEM((BS, NB), jnp.int32),
            pltpu.VMEM((BS, NB), jnp.float32),
            pltpu.VMEM((NB, d), jnp.float32),
            pltpu.VMEM((NB, d), jnp.float32),
            pltpu.VMEM((NB,), jnp.float32),
            pltpu.VMEM((CB, d), jnp.float32),
            pltpu.VMEM((rpt,), jnp.float32),
            pltpu.VMEM_SHARED((npad, d), jnp.float32),
            pltpu.VMEM_SHARED((npad,), jnp.float32),
            pltpu.SemaphoreType.DMA,
            pltpu.SemaphoreType.DMA,
            pltpu.SemaphoreType.DMA,
        ],
    )
    def stage1(x_h, src_h, dst_h, k_h, h1_h, inv_h,
               src_v, dst_v, k_v, rows0, rows1, ones_v, aggbuf, invbuf,
               agg_s, deg_s, gsem, ssem0, ssem1):
        c = lax.axis_index("c")
        s = lax.axis_index("s")
        r0 = s * rpt

        _fill1(ones_v, NB, 1.0)
        _zero2(aggbuf, CB, d)
        for cb in range(rpt // CB):
            pltpu.sync_copy(aggbuf, agg_s.at[pl.ds(r0 + cb * CB, CB)])
        _fill1(invbuf, rpt, 0.0)
        pltpu.sync_copy(invbuf, deg_s.at[pl.ds(r0, rpt)])
        plsc.subcore_barrier()

        _edge_pass(x_h, src_h, dst_h, k_h, c, s, src_v, dst_v, k_v,
                   (rows0, rows1), gsem, (ssem0, ssem1), agg_s, nblk, d,
                   ones_v=ones_v, deg_s=deg_s)
        plsc.subcore_barrier()

        pltpu.sync_copy(deg_s.at[pl.ds(r0, rpt)], invbuf)

        def inv(i, _):
            dv = invbuf[pl.ds(i * LANES, LANES)]
            invbuf[pl.ds(i * LANES, LANES)] = 1.0 / jnp.maximum(dv, 1.0)
            return None
        lax.fori_loop(0, rpt // LANES, inv, None)

        @pl.when(c == 0)
        def _():
            pltpu.sync_copy(invbuf, inv_h.at[pl.ds(r0, rpt)])

        _copyout(agg_s, aggbuf, invbuf, h1_h.at[c], r0, rpt, d)

    return stage1


def _make_stage2(npad, d, nblk):
    rpt = npad // NTILE
    mesh = plsc.VectorSubcoreMesh(core_axis_name="c", subcore_axis_name="s",
                                  num_cores=NSC, num_subcores=NTILE)

    @functools.partial(
        pl.kernel,
        out_type=jax.ShapeDtypeStruct((NSC, 2, npad, d), jnp.float32),
        mesh=mesh,
        scratch_types=[
            pltpu.VMEM((BS, NB), jnp.int32),
            pltpu.VMEM((BS, NB), jnp.int32),
            pltpu.VMEM((BS, NB), jnp.float32),
            pltpu.VMEM((NB, d), jnp.float32),
            pltpu.VMEM((NB, d), jnp.float32),
            pltpu.VMEM((CB, d), jnp.float32),
            pltpu.VMEM((rpt,), jnp.float32),
            pltpu.VMEM_SHARED((npad, d), jnp.float32),
            pltpu.SemaphoreType.DMA,
            pltpu.SemaphoreType.DMA,
            pltpu.SemaphoreType.DMA,
        ],
    )
    def stage2(t0_h, t1_h, src_h, dst_h, k_h, inv_h, q_h,
               src_v, dst_v, k_v, rows0, rows1, aggbuf, invbuf, agg_s,
               gsem, ssem0, ssem1):
        c = lax.axis_index("c")
        s = lax.axis_index("s")
        r0 = s * rpt

        pltpu.sync_copy(inv_h.at[pl.ds(r0, rpt)], invbuf)

        for p, table_h in enumerate((t0_h, t1_h)):
            _zero2(aggbuf, CB, d)
            for cb in range(rpt // CB):
                pltpu.sync_copy(aggbuf, agg_s.at[pl.ds(r0 + cb * CB, CB)])
            plsc.subcore_barrier()

            _edge_pass(table_h, src_h, dst_h, k_h, c, s, src_v, dst_v, k_v,
                       (rows0, rows1), gsem, (ssem0, ssem1), agg_s, nblk, d)
            plsc.subcore_barrier()

            _copyout(agg_s, aggbuf, invbuf, q_h.at[c, p], r0, rpt, d)

    return stage2


def _mlp_block(q0, q1, q2, q3, w1, b1, w2, b2, o_ref):
    d = q0.shape[1]
    acc = jnp.dot(q0[...], w1[0:d, :], preferred_element_type=jnp.float32)
    acc += jnp.dot(q1[...], w1[d:2 * d, :], preferred_element_type=jnp.float32)
    acc += jnp.dot(q2[...], w1[2 * d:3 * d, :],
                   preferred_element_type=jnp.float32)
    acc += jnp.dot(q3[...], w1[3 * d:4 * d, :],
                   preferred_element_type=jnp.float32)
    h = jnp.maximum(acc + b1[...], 0.0)
    o = jnp.dot(h, w2[...], preferred_element_type=jnp.float32) + b2[...]
    ss = jnp.sum(o * o, axis=-1, keepdims=True)
    o_ref[...] = o / jnp.maximum(jnp.sqrt(ss), 1e-12)


def _make_mlp(npad, d, hidden, out):
    rb = 640
    grid = npad // rb
    return pl.pallas_call(
        _mlp_block,
        grid=(grid,),
        in_specs=[
            pl.BlockSpec((rb, d), lambda i: (i, 0)),
            pl.BlockSpec((rb, d), lambda i: (i, 0)),
            pl.BlockSpec((rb, d), lambda i: (i, 0)),
            pl.BlockSpec((rb, d), lambda i: (i, 0)),
            pl.BlockSpec((4 * d, hidden), lambda i: (0, 0)),
            pl.BlockSpec((1, hidden), lambda i: (0, 0)),
            pl.BlockSpec((hidden, out), lambda i: (0, 0)),
            pl.BlockSpec((1, out), lambda i: (0, 0)),
        ],
        out_specs=pl.BlockSpec((rb, out), lambda i: (i, 0)),
        out_shape=jax.ShapeDtypeStruct((npad, out), jnp.float32),
    )


@jax.jit
def kernel(x, edge_index, K_DD, K_DA, W1, b1, W2, b2):
    n, d = x.shape
    e = edge_index.shape[1]
    hidden = W1.shape[1]
    out = W2.shape[1]
    npad = ((n + 256) // 256) * 256

    ept = NB * BS * -(-e // (NB * BS * NTILE))
    nblk = ept // (NB * BS)
    e_pad = ept * NTILE
    pad = e_pad - e
    src = jnp.pad(edge_index[0], (0, pad)).reshape(NTILE, nblk, BS, NB)
    dst = jnp.pad(edge_index[1], (0, pad),
                  constant_values=n).reshape(NTILE, nblk, BS, NB)
    kdd = jnp.pad(K_DD, ((0, 0), (0, pad))).reshape(NSC, NTILE, nblk, BS, NB)
    kda = jnp.pad(K_DA, ((0, 0), (0, pad))).reshape(NSC, NTILE, nblk, BS, NB)

    stage1 = _make_stage1(npad, d, nblk)
    h1, inv = stage1(x, src, dst, kdd)

    stage2 = _make_stage2(npad, d, nblk)
    q = stage2(h1[0], h1[1], src, dst, kda, inv)

    mlp = _make_mlp(npad, d, hidden, out)
    res = mlp(q[0, 0], q[0, 1], q[1, 0], q[1, 1], W1,
              b1.reshape(1, hidden), W2, b2.reshape(1, out))
    return res[:n]


# speedup vs baseline: 2.7190x; 1.1464x over previous
import functools

import jax
import jax.numpy as jnp
from jax import lax
from jax.experimental import pallas as pl
from jax.experimental.pallas import tpu as pltpu
from jax.experimental.pallas import tpu_sc as plsc

NB = 128
BS = 8
NSC = 2
NTILE = 16
LANES = 16
CB = 32


def _fill1(ref, n, val):
    def body(i, _):
        ref[pl.ds(i * LANES, LANES)] = jnp.full((LANES,), val, jnp.float32)
        return None
    lax.fori_loop(0, n // LANES, body, None)


def _fill2(ref, nrows, ncols, val):
    def body(i, _):
        for j in range(ncols // LANES):
            ref[i, pl.ds(j * LANES, LANES)] = jnp.full((LANES,), val,
                                                       jnp.float32)
        return None
    lax.fori_loop(0, nrows, body, None)


def _splat(vec, lane):
    return vec.at[jnp.full((LANES,), lane, jnp.int32)].get(
        mode="promise_in_bounds")


def _edge_pass(table_h, src_h, dst_h, k_h, c, s, src_v, dst_v, k_v,
               rows2, msg, gsem, ssem, dsem, agg_s, nblk, d,
               ones_v=None, deg_s=None):
    def mul(rows_bf, ch):
        def body(e, _):
            base = (e // LANES) * LANES
            wvec = k_v[ch, pl.ds(base, LANES)]
            kv = _splat(wvec, e - base)
            for j in range(d // (2 * LANES)):
                w32 = rows_bf[e, pl.ds(j * LANES, LANES)]
                a = plsc.bitcast(w32 << 16, jnp.float32)
                b = plsc.bitcast(w32 & jnp.int32(-65536), jnp.float32)
                msg[e, pl.ds(j * 2 * LANES, LANES)] = a * kv
                msg[e, pl.ds(j * 2 * LANES + LANES, LANES)] = b * kv
            return None
        lax.fori_loop(0, NB, body, None)

    def block(bo, _):
        pltpu.sync_copy(src_h.at[s, bo], src_v)
        pltpu.sync_copy(dst_h.at[s, bo], dst_v)
        pltpu.sync_copy(k_h.at[c, s, bo], k_v)

        gd = pltpu.async_copy(table_h.at[src_v.at[0]], rows2[0], gsem)
        sd = None
        dd = []
        for ch in range(BS):
            b = ch & 1
            gd.wait()
            if ch + 1 < BS:
                gd = pltpu.async_copy(table_h.at[src_v.at[ch + 1]],
                                      rows2[1 - b], gsem)
            if sd is not None:
                sd.wait()
            mul(rows2[b], ch)
            sd = pltpu.async_copy(msg, agg_s.at[dst_v.at[ch]], ssem,
                                  add=True)
            if deg_s is not None:
                dd.append(pltpu.async_copy(ones_v, deg_s.at[dst_v.at[ch]],
                                           dsem, add=True))
        sd.wait()
        for dde in dd:
            dde.wait()
        return None

    lax.fori_loop(0, nblk, block, None)


def _make_stage1(npad, d, nblk):
    rpt = npad // NTILE
    mesh = plsc.VectorSubcoreMesh(core_axis_name="c", subcore_axis_name="s",
                                  num_cores=NSC, num_subcores=NTILE)

    @functools.partial(
        pl.kernel,
        out_type=(jax.ShapeDtypeStruct((NSC, npad, d // 2), jnp.int32),
                  jax.ShapeDtypeStruct((npad,), jnp.float32)),
        mesh=mesh,
        compiler_params=pltpu.CompilerParams(
            needs_layout_passes=False, use_tc_tiling_on_sc=False),
        scratch_types=[
            pltpu.VMEM((BS, NB), jnp.int32),
            pltpu.VMEM((BS, NB), jnp.int32),
            pltpu.VMEM((BS, NB), jnp.float32),
            pltpu.VMEM((NB, 64), jnp.int32),
            pltpu.VMEM((NB, 64), jnp.int32),
            pltpu.VMEM((NB, 128), jnp.float32),
            pltpu.VMEM((NB,), jnp.float32),
            pltpu.VMEM((CB, 128), jnp.float32),
            pltpu.VMEM((CB, 64), jnp.int32),
            pltpu.VMEM((rpt,), jnp.float32),
            pltpu.VMEM_SHARED((npad, 128), jnp.float32),
            pltpu.VMEM_SHARED((npad,), jnp.float32),
            pltpu.SemaphoreType.DMA,
            pltpu.SemaphoreType.DMA,
            pltpu.SemaphoreType.DMA,
        ],
    )
    def stage1(x_h, src_h, dst_h, k_h, h1_h, inv_h,
               src_v, dst_v, k_v, rows0, rows1, msg, ones_v, aggbuf, packb,
               invbuf, agg_s, deg_s, gsem, ssem, dsem):
        c = lax.axis_index("c")
        s = lax.axis_index("s")
        r0 = s * rpt
        d = 128

        _fill1(ones_v, NB, 1.0)
        _fill2(aggbuf, CB, d, 0.0)
        for cb in range(rpt // CB):
            pltpu.sync_copy(aggbuf, agg_s.at[pl.ds(r0 + cb * CB, CB)])
        _fill1(invbuf, rpt, 0.0)
        pltpu.sync_copy(invbuf, deg_s.at[pl.ds(r0, rpt)])
        plsc.subcore_barrier()

        _edge_pass(x_h, src_h, dst_h, k_h, c, s, src_v, dst_v, k_v,
                   (rows0, rows1), msg, gsem, ssem, dsem, agg_s, nblk, d,
                   ones_v=ones_v, deg_s=deg_s)
        plsc.subcore_barrier()

        pltpu.sync_copy(deg_s.at[pl.ds(r0, rpt)], invbuf)

        def inv(i, _):
            dv = invbuf[pl.ds(i * LANES, LANES)]
            invbuf[pl.ds(i * LANES, LANES)] = 1.0 / jnp.maximum(dv, 1.0)
            return None
        lax.fori_loop(0, rpt // LANES, inv, None)

        @pl.when(c == 0)
        def _():
            pltpu.sync_copy(invbuf, inv_h.at[pl.ds(r0, rpt)])

        for cb in range(rpt // CB):
            base = r0 + cb * CB
            pltpu.sync_copy(agg_s.at[pl.ds(base, CB)], aggbuf)

            def row(r, _, cb=cb):
                rb = (r // LANES) * LANES
                ivec = invbuf[pl.ds(cb * CB + rb, LANES)]
                iv = _splat(ivec, r - rb)
                for j in range(d // (2 * LANES)):
                    a = aggbuf[r, pl.ds(j * 2 * LANES, LANES)] * iv
                    b = aggbuf[r, pl.ds(j * 2 * LANES + LANES, LANES)] * iv
                    ai = plsc.bitcast(a, jnp.int32)
                    bi = plsc.bitcast(b, jnp.int32)
                    ra = lax.shift_right_logical(
                        ai + 32767 + (lax.shift_right_logical(ai, 16) & 1),
                        16)
                    rb = lax.shift_right_logical(
                        bi + 32767 + (lax.shift_right_logical(bi, 16) & 1),
                        16)
                    packb[r, pl.ds(j * LANES, LANES)] = ra | (rb << 16)
                return None
            lax.fori_loop(0, CB, row, None)
            pltpu.sync_copy(packb, h1_h.at[c, pl.ds(base, CB)])

    return stage1


def _make_stage2(npad, d, nblk):
    rpt = npad // NTILE
    mesh = plsc.VectorSubcoreMesh(core_axis_name="c", subcore_axis_name="s",
                                  num_cores=NSC, num_subcores=NTILE)

    @functools.partial(
        pl.kernel,
        out_type=jax.ShapeDtypeStruct((NSC, 2, npad, d), jnp.float32),
        mesh=mesh,
        compiler_params=pltpu.CompilerParams(
            needs_layout_passes=False, use_tc_tiling_on_sc=False),
        scratch_types=[
            pltpu.VMEM((BS, NB), jnp.int32),
            pltpu.VMEM((BS, NB), jnp.int32),
            pltpu.VMEM((BS, NB), jnp.float32),
            pltpu.VMEM((NB, 64), jnp.int32),
            pltpu.VMEM((NB, 64), jnp.int32),
            pltpu.VMEM((NB, 128), jnp.float32),
            pltpu.VMEM((CB, 128), jnp.float32),
            pltpu.VMEM((rpt,), jnp.float32),
            pltpu.VMEM_SHARED((npad, 128), jnp.float32),
            pltpu.SemaphoreType.DMA,
            pltpu.SemaphoreType.DMA,
            pltpu.SemaphoreType.DMA,
        ],
    )
    def stage2(t0_h, t1_h, src_h, dst_h, k_h, inv_h, q_h,
               src_v, dst_v, k_v, rows0, rows1, msg, aggbuf, invbuf, agg_s,
               gsem, ssem, dsem):
        c = lax.axis_index("c")
        s = lax.axis_index("s")
        r0 = s * rpt
        d = 128

        pltpu.sync_copy(inv_h.at[pl.ds(r0, rpt)], invbuf)

        for p, table_h in enumerate((t0_h, t1_h)):
            _fill2(aggbuf, CB, d, 0.0)
            for cb in range(rpt // CB):
                pltpu.sync_copy(aggbuf, agg_s.at[pl.ds(r0 + cb * CB, CB)])
            plsc.subcore_barrier()

            _edge_pass(table_h, src_h, dst_h, k_h, c, s, src_v, dst_v, k_v,
                       (rows0, rows1), msg, gsem, ssem, dsem, agg_s, nblk, d)
            plsc.subcore_barrier()

            for cb in range(rpt // CB):
                base = r0 + cb * CB
                pltpu.sync_copy(agg_s.at[pl.ds(base, CB)], aggbuf)

                def row(r, _, cb=cb):
                    rb = (r // LANES) * LANES
                    ivec = invbuf[pl.ds(cb * CB + rb, LANES)]
                    iv = _splat(ivec, r - rb)
                    for j in range(d // LANES):
                        aggbuf[r, pl.ds(j * LANES, LANES)] = (
                            aggbuf[r, pl.ds(j * LANES, LANES)] * iv)
                    return None
                lax.fori_loop(0, CB, row, None)
                pltpu.sync_copy(aggbuf, q_h.at[c, p, pl.ds(base, CB)])

    return stage2


def _mlp_block(q0, q1, q2, q3, w1, b1, w2, b2, o_ref):
    d = q0.shape[1]
    acc = jnp.dot(q0[...], w1[0:d, :], preferred_element_type=jnp.float32)
    acc += jnp.dot(q1[...], w1[d:2 * d, :], preferred_element_type=jnp.float32)
    acc += jnp.dot(q2[...], w1[2 * d:3 * d, :],
                   preferred_element_type=jnp.float32)
    acc += jnp.dot(q3[...], w1[3 * d:4 * d, :],
                   preferred_element_type=jnp.float32)
    h = jnp.maximum(acc + b1[...], 0.0)
    o = jnp.dot(h, w2[...], preferred_element_type=jnp.float32) + b2[...]
    ss = jnp.sum(o * o, axis=-1, keepdims=True)
    o_ref[...] = o / jnp.maximum(jnp.sqrt(ss), 1e-12)


def _make_mlp(npad, d, hidden, out):
    rb = 640
    grid = npad // rb
    return pl.pallas_call(
        _mlp_block,
        grid=(grid,),
        in_specs=[
            pl.BlockSpec((rb, d), lambda i: (i, 0)),
            pl.BlockSpec((rb, d), lambda i: (i, 0)),
            pl.BlockSpec((rb, d), lambda i: (i, 0)),
            pl.BlockSpec((rb, d), lambda i: (i, 0)),
            pl.BlockSpec((4 * d, hidden), lambda i: (0, 0)),
            pl.BlockSpec((1, hidden), lambda i: (0, 0)),
            pl.BlockSpec((hidden, out), lambda i: (0, 0)),
            pl.BlockSpec((1, out), lambda i: (0, 0)),
        ],
        out_specs=pl.BlockSpec((rb, out), lambda i: (i, 0)),
        out_shape=jax.ShapeDtypeStruct((npad, out), jnp.float32),
    )


def _interleave_cols(a):
    n, d = a.shape
    g = a.reshape(n, d // 32, 2, 16)
    return g.transpose(0, 1, 3, 2).reshape(n, d)


@jax.jit
def kernel(x, edge_index, K_DD, K_DA, W1, b1, W2, b2):
    n, d = x.shape
    e = edge_index.shape[1]
    hidden = W1.shape[1]
    out = W2.shape[1]
    npad = ((n + 256) // 256) * 256

    ept = NB * BS * -(-e // (NB * BS * NTILE))
    nblk = ept // (NB * BS)
    e_pad = ept * NTILE
    pad = e_pad - e
    src = jnp.pad(edge_index[0], (0, pad)).reshape(NTILE, nblk, BS, NB)
    dst = jnp.pad(edge_index[1], (0, pad),
                  constant_values=n).reshape(NTILE, nblk, BS, NB)
    kdd = jnp.pad(K_DD, ((0, 0), (0, pad))).reshape(NSC, NTILE, nblk, BS, NB)
    kda = jnp.pad(K_DA, ((0, 0), (0, pad))).reshape(NSC, NTILE, nblk, BS, NB)

    x_bf = jax.lax.bitcast_convert_type(
        _interleave_cols(x).astype(jnp.bfloat16).reshape(n, d // 2, 2),
        jnp.int32)

    stage1 = _make_stage1(npad, d, nblk)
    h1, inv = stage1(x_bf, src, dst, kdd)

    stage2 = _make_stage2(npad, d, nblk)
    q = stage2(h1[0], h1[1], src, dst, kda, inv)

    mlp = _make_mlp(npad, d, hidden, out)
    res = mlp(q[0, 0], q[0, 1], q[1, 0], q[1, 1], W1,
              b1.reshape(1, hidden), W2, b2.reshape(1, out))
    return res[:n]
